# baseline (speedup 1.0000x reference)
import dataclasses
import functools

import jax
import jax.numpy as jnp
from jax import lax
from jax.experimental import pallas as pl
from jax.experimental.pallas import tpu as pltpu
from jax.experimental.pallas import tpu_sc as plsc

_B = 16384
_C = 1000
_BC = 1024
_NB = _B // _BC

_NEG = float(-jnp.inf)


def _dlr_body(xt_ref, tl_ref, out_ref):
    yv = tl_ref[0]

    gs = [xt_ref[k * 128:(k + 1) * 128, :] for k in range(7)]
    gs.append(jnp.concatenate(
        [xt_ref[896:1000, :], jnp.full((24, _BC), _NEG, jnp.float32)],
        axis=0))

    hi = [jnp.maximum(gs[2 * i], gs[2 * i + 1]) for i in range(4)]
    lo = [jnp.minimum(gs[2 * i], gs[2 * i + 1]) for i in range(4)]

    def top3of4(a, b, c, d):
        p1 = jnp.maximum(a, c)
        p2 = jnp.minimum(a, c)
        q1 = jnp.maximum(b, d)
        return p1, jnp.maximum(p2, q1), jnp.minimum(p2, q1)

    x1, x2, x3 = top3of4(hi[0], lo[0], hi[1], lo[1])
    y1, y2, y3 = top3of4(hi[2], lo[2], hi[3], lo[3])
    z1 = jnp.maximum(x1, y1)
    m11 = jnp.minimum(x1, y1)
    m22 = jnp.maximum(x2, y2)
    z2 = jnp.maximum(m11, m22)
    z3 = jnp.maximum(jnp.maximum(jnp.minimum(m22, m11), jnp.minimum(x2, y2)),
                     jnp.maximum(x3, y3))

    m1 = jnp.max(z1, axis=0, keepdims=True)
    c1 = z1 == m1
    cnt1 = jnp.sum(jnp.where(c1, 1.0, 0.0), axis=0, keepdims=True)
    b = jnp.where(c1, z2, z1)
    m2r = jnp.max(b, axis=0, keepdims=True)
    cb = b == m2r
    cnt2 = jnp.sum(jnp.where(cb, 1.0, 0.0), axis=0, keepdims=True)
    c = jnp.where(cb, jnp.where(c1, z3, z2), b)
    m3r = jnp.max(c, axis=0, keepdims=True)
    m2 = jnp.where(cnt1 > 1.0, m1, m2r)
    m3 = jnp.where(cnt1 > 2.0, m1,
                   jnp.where(cnt1 > 1.0, m2r,
                             jnp.where(cnt2 > 1.0, m2r, m3r)))

    tl = yv

    ind = tl == m1
    dividend = tl - jnp.where(ind, m2, m1)
    divisor = m1 - m3 + 1e-12
    out_ref[0] = jnp.sum(dividend / divisor, axis=1, keepdims=True)


_NW = 32
_PER = _B // _NW
_DG = 8


def _tl_gather_sc(xt, y):
    mesh = plsc.VectorSubcoreMesh(core_axis_name="c", subcore_axis_name="s")
    cp_sc = pltpu.CompilerParams()
    if "needs_layout_passes" in pltpu.CompilerParams.__dataclass_fields__:
        cp_sc = dataclasses.replace(cp_sc, needs_layout_passes=False)

    @functools.partial(
        pl.kernel, mesh=mesh, compiler_params=cp_sc,
        out_type=jax.ShapeDtypeStruct((_B,), jnp.float32),
        scratch_types=[
            pltpu.VMEM((_PER,), jnp.int32),
            pltpu.VMEM((_PER // 128, 128, 128), jnp.float32),
            pltpu.VMEM((_PER,), jnp.float32),
            pltpu.SemaphoreType.DMA,
        ],
    )
    def k(xt_hbm, y_hbm, tl_hbm, y_v, seg_v, tl_v, sem):
        wid = lax.axis_index("s") * 2 + lax.axis_index("c")
        base = wid * _PER
        pltpu.sync_copy(y_hbm.at[pl.ds(base, _PER)], y_v)

        rows16 = lax.iota(jnp.int32, 16)
        nw = _PER // 128

        copies = []
        for w in range(nw):
            idxs = y_v.at[pl.ds(w * 128, 128)]
            copies.append(pltpu.async_copy(
                xt_hbm.at[idxs, pl.ds(base + w * 128, 128)],
                seg_v.at[w], sem))
        for w in range(nw):
            copies[w].wait()
            for h in range(8):
                r16 = rows16 + 16 * h
                vals = plsc.load_gather(seg_v.at[w], [r16, r16])
                tl_v[pl.ds(w * 128 + h * 16, 16)] = vals

        pltpu.sync_copy(tl_v, tl_hbm.at[pl.ds(base, _PER)])

    return k(xt, y)


def kernel(x, y):
    xt = x.T
    tl = _tl_gather_sc(xt, y.astype(jnp.int32))
    tl3 = tl.reshape(_NB, 1, _BC)
    partial = pl.pallas_call(
        _dlr_body,
        grid=(_NB,),
        in_specs=[
            pl.BlockSpec((_C, _BC), lambda i: (0, i)),
            pl.BlockSpec((1, 1, _BC), lambda i: (i, 0, 0)),
        ],
        out_specs=pl.BlockSpec((1, 1, 1), lambda i: (i, 0, 0)),
        out_shape=jax.ShapeDtypeStruct((_NB, 1, 1), jnp.float32),
        compiler_params=pltpu.CompilerParams(
            dimension_semantics=("parallel",),
        ),
    )(xt, tl3)
    return -(jnp.sum(partial) / _B)


# speedup vs baseline: 1.6083x; 1.6083x over previous
import jax
import jax.numpy as jnp
from jax.experimental import pallas as pl
from jax.experimental.pallas import tpu as pltpu

_B = 16384
_C = 1000
_BC = 1024
_NB = _B // _BC

_NEG = float(-jnp.inf)


def _dlr_body(xt_ref, y_ref, out_ref):
    yv = y_ref[0]

    gs = [xt_ref[k * 128:(k + 1) * 128, :] for k in range(7)]
    gs.append(jnp.concatenate(
        [xt_ref[896:1000, :], jnp.full((24, _BC), _NEG, jnp.float32)],
        axis=0))

    hi = [jnp.maximum(gs[2 * i], gs[2 * i + 1]) for i in range(4)]
    lo = [jnp.minimum(gs[2 * i], gs[2 * i + 1]) for i in range(4)]

    def top3of4(a, b, c, d):
        p1 = jnp.maximum(a, c)
        p2 = jnp.minimum(a, c)
        q1 = jnp.maximum(b, d)
        return p1, jnp.maximum(p2, q1), jnp.minimum(p2, q1)

    x1, x2, x3 = top3of4(hi[0], lo[0], hi[1], lo[1])
    y1, y2, y3 = top3of4(hi[2], lo[2], hi[3], lo[3])
    z1 = jnp.maximum(x1, y1)
    m11 = jnp.minimum(x1, y1)
    m22 = jnp.maximum(x2, y2)
    z2 = jnp.maximum(m11, m22)
    z3 = jnp.maximum(jnp.maximum(jnp.minimum(m22, m11), jnp.minimum(x2, y2)),
                     jnp.maximum(x3, y3))

    m1 = jnp.max(z1, axis=0, keepdims=True)
    c1 = z1 == m1
    cnt1 = jnp.sum(jnp.where(c1, 1.0, 0.0), axis=0, keepdims=True)
    b = jnp.where(c1, z2, z1)
    m2r = jnp.max(b, axis=0, keepdims=True)
    cb = b == m2r
    cnt2 = jnp.sum(jnp.where(cb, 1.0, 0.0), axis=0, keepdims=True)
    c = jnp.where(cb, jnp.where(c1, z3, z2), b)
    m3r = jnp.max(c, axis=0, keepdims=True)
    m2 = jnp.where(cnt1 > 1.0, m1, m2r)
    m3 = jnp.where(cnt1 > 2.0, m1,
                   jnp.where(cnt1 > 1.0, m2r,
                             jnp.where(cnt2 > 1.0, m2r, m3r)))

    sid = jax.lax.shift_right_logical(yv, 7)
    b0 = (sid & 1) == 1
    b1 = (jax.lax.shift_right_logical(sid, 1) & 1) == 1
    b2 = jax.lax.shift_right_logical(sid, 2) == 1
    t01 = jnp.where(b0, gs[1], gs[0])
    t23 = jnp.where(b0, gs[3], gs[2])
    t45 = jnp.where(b0, gs[5], gs[4])
    t67 = jnp.where(b0, gs[7], gs[6])
    t03 = jnp.where(b1, t23, t01)
    t47 = jnp.where(b1, t67, t45)
    g_y = jnp.where(b2, t47, t03)
    rows = jax.lax.broadcasted_iota(jnp.int32, (128, _BC), 0)
    oh = rows == (yv & 127)
    tl = jnp.max(jnp.where(oh, g_y, _NEG), axis=0, keepdims=True)

    ind = tl == m1
    dividend = tl - jnp.where(ind, m2, m1)
    divisor = m1 - m3 + 1e-12
    out_ref[0] = jnp.sum(dividend / divisor, axis=1, keepdims=True)


def kernel(x, y):
    xt = x.T
    y3 = y.astype(jnp.int32).reshape(_NB, 1, _BC)
    partial = pl.pallas_call(
        _dlr_body,
        grid=(_NB,),
        in_specs=[
            pl.BlockSpec((_C, _BC), lambda i: (0, i)),
            pl.BlockSpec((1, 1, _BC), lambda i: (i, 0, 0)),
        ],
        out_specs=pl.BlockSpec((1, 1, 1), lambda i: (i, 0, 0)),
        out_shape=jax.ShapeDtypeStruct((_NB, 1, 1), jnp.float32),
        compiler_params=pltpu.CompilerParams(
            dimension_semantics=("parallel",),
        ),
    )(xt, y3)
    return -(jnp.sum(partial) / _B)
